# baseline (speedup 1.0000x reference)
import functools

import jax
import jax.numpy as jnp
from jax import lax
from jax.experimental import pallas as pl
from jax.experimental.pallas import tpu as pltpu
from jax.experimental.pallas import tpu_sc as plsc

D_MODEL = 64
SUB = 128
K = 8


def _sc_gather(idx2d, table, total):
    info = plsc.get_sparse_core_info()
    nw = info.num_cores * info.num_subcores
    n_rows = idx2d.shape[0]
    rows_per_w = n_rows // nw
    n_chunks = rows_per_w // K
    chunk = K * SUB
    mesh = plsc.VectorSubcoreMesh(core_axis_name="c", subcore_axis_name="s")

    @functools.partial(
        pl.kernel,
        mesh=mesh,
        out_type=jax.ShapeDtypeStruct((total, D_MODEL), jnp.float32),
        scratch_types=[
            pltpu.VMEM((K, SUB), jnp.int32),
            pltpu.VMEM((chunk, D_MODEL), jnp.float32),
            pltpu.SemaphoreType.DMA,
        ],
        compiler_params=pltpu.CompilerParams(use_tc_tiling_on_sc=False),
    )
    def k(idx_hbm, table_hbm, out_hbm, idx_v, rows_v, sem):
        wid = lax.axis_index("s") * info.num_cores + lax.axis_index("c")
        row0 = wid * rows_per_w

        def body(j, carry):
            r0 = row0 + j * K
            pltpu.sync_copy(idx_hbm.at[pl.ds(r0, K)], idx_v)
            copies = [
                pltpu.async_copy(
                    table_hbm.at[idx_v.at[r]],
                    rows_v.at[pl.ds(r * SUB, SUB)],
                    sem,
                )
                for r in range(K)
            ]
            for c in copies:
                c.wait()
            pltpu.sync_copy(rows_v, out_hbm.at[pl.ds(r0 * SUB, chunk)])
            return carry

        lax.fori_loop(0, n_chunks, body, 0)

    return k(idx2d, table)


def kernel(action_tokens, table):
    b, s = action_tokens.shape
    total = b * s
    idx2d = action_tokens.reshape(total // SUB, SUB).astype(jnp.int32)
    out = _sc_gather(idx2d, table, total)
    return out.reshape(b, s, D_MODEL)


# speedup vs baseline: 1.0205x; 1.0205x over previous
import functools

import jax
import jax.numpy as jnp
from jax import lax
from jax.experimental import pallas as pl
from jax.experimental.pallas import tpu as pltpu
from jax.experimental.pallas import tpu_sc as plsc

D_MODEL = 64
SUB = 128
K = 2
NBUF = 4
CHUNK = K * SUB


def _sc_gather(idx2d, table, total):
    info = plsc.get_sparse_core_info()
    nw = info.num_cores * info.num_subcores
    n_rows = idx2d.shape[0]
    rows_per_w = n_rows // nw
    n_chunks = rows_per_w // K
    n_iters = n_chunks // NBUF
    mesh = plsc.VectorSubcoreMesh(core_axis_name="c", subcore_axis_name="s")

    @functools.partial(
        pl.kernel,
        mesh=mesh,
        out_type=jax.ShapeDtypeStruct((total, D_MODEL), jnp.float32),
        scratch_types=[
            pltpu.VMEM((rows_per_w, SUB), jnp.int32),
            [pltpu.VMEM((CHUNK, D_MODEL), jnp.float32) for _ in range(NBUF)],
            [pltpu.SemaphoreType.DMA for _ in range(NBUF)],
            [pltpu.SemaphoreType.DMA for _ in range(NBUF)],
        ],
        compiler_params=pltpu.CompilerParams(use_tc_tiling_on_sc=False),
    )
    def k(idx_hbm, table_hbm, out_hbm, idx_all, rows, gsem, wsem):
        wid = lax.axis_index("s") * info.num_cores + lax.axis_index("c")
        base = wid * rows_per_w * SUB

        pltpu.sync_copy(idx_hbm.at[pl.ds(wid * rows_per_w, rows_per_w)],
                        idx_all)

        def fire_g(b, j):
            for r in range(K):
                pltpu.async_copy(
                    table_hbm.at[idx_all.at[j * K + r]],
                    rows[b].at[pl.ds(r * SUB, SUB)],
                    gsem[b])

        def wait_g(b, j):
            for r in range(K):
                pltpu.make_async_copy(
                    table_hbm.at[idx_all.at[j * K + r]],
                    rows[b].at[pl.ds(r * SUB, SUB)],
                    gsem[b]).wait()

        def fire_w(b, j):
            pltpu.async_copy(
                rows[b], out_hbm.at[pl.ds(base + j * CHUNK, CHUNK)], wsem[b])

        def wait_w(b):
            pltpu.make_async_copy(
                rows[b], out_hbm.at[pl.ds(base, CHUNK)], wsem[b]).wait()

        fire_g(0, 0)
        for b in range(1, NBUF):
            fire_g(b, b)
            wait_g(b - 1, b - 1)
            fire_w(b - 1, b - 1)

        def body(t, carry):
            j0 = t * NBUF
            for b in range(NBUF):
                pb = (b - 1) % NBUF
                wait_w(b)
                fire_g(b, j0 + b)
                wait_g(pb, j0 + b - 1)
                fire_w(pb, j0 + b - 1)
            return carry

        lax.fori_loop(1, n_iters, body, 0)

        last = n_chunks - 1
        wait_g(NBUF - 1, last)
        fire_w(NBUF - 1, last)
        for b in range(NBUF):
            wait_w(b)

    return k(idx2d, table)


def kernel(action_tokens, table):
    b, s = action_tokens.shape
    total = b * s
    idx2d = action_tokens.reshape(total // SUB, SUB).astype(jnp.int32)
    out = _sc_gather(idx2d, table, total)
    return out.reshape(b, s, D_MODEL)
